# baseline (speedup 1.0000x reference)
import functools

import jax
import jax.numpy as jnp
from jax import lax
from jax.experimental import pallas as pl
from jax.experimental.pallas import tpu as pltpu
from jax.experimental.pallas import tpu_sc as plsc

HALF = 192
NEMB = 32
NW = 32
SLAB = 128
NLANE = 16
RPT = (NEMB * NEMB) // 16


def _build_sc_call(s, s_pad):
    n_units = s * 2
    kmax = -(-n_units // NW)
    mesh = plsc.VectorSubcoreMesh(core_axis_name="c", subcore_axis_name="s")

    @functools.partial(
        pl.kernel,
        mesh=mesh,
        out_type=jax.ShapeDtypeStruct((s, s, 2 * HALF), jnp.float32),
        scratch_types=[
            pltpu.VMEM((2 * HALF,), jnp.float32),
            pltpu.VMEM((NEMB * HALF,), jnp.float32),
            pltpu.VMEM((1, s_pad), jnp.int32),
            pltpu.VMEM((1, s_pad), jnp.int32),
            pltpu.VMEM((SLAB,), jnp.int32),
            pltpu.VMEM((2, SLAB, 2 * HALF), jnp.float32),
            pltpu.VMEM_SHARED((NEMB * NEMB, 2 * HALF), jnp.float32),
            pltpu.SemaphoreType.DMA,
            pltpu.SemaphoreType.DMA,
        ],
    )
    def sc_fn(xt_hbm, yt_hbm, xd_hbm, yd_hbm, out_hbm, xt_v, yt_v,
              xd_st, yd_st, idx_v, rows, comb_sp, sem_g, sem_w):
        sid = lax.axis_index("s")
        wid = sid * 2 + lax.axis_index("c")
        pltpu.sync_copy(xt_hbm.at[pl.ds(sid * 2 * HALF, 2 * HALF)], xt_v)
        pltpu.sync_copy(yt_hbm, yt_v)

        def brow(i, carry):
            a = i // NEMB
            b = i % NEMB
            for k in range(HALF // NLANE):
                rows[0, i, pl.ds(k * NLANE, NLANE)] = (
                    xt_v[pl.ds(a * HALF + k * NLANE, NLANE)])
                rows[0, i, pl.ds(HALF + k * NLANE, NLANE)] = (
                    yt_v[pl.ds(b * HALF + k * NLANE, NLANE)])
            return carry

        lax.fori_loop(0, RPT, brow, 0)
        pltpu.sync_copy(rows.at[0, pl.ds(0, RPT)],
                        comb_sp.at[pl.ds(sid * RPT, RPT)])
        plsc.subcore_barrier()

        def fill(j0, buf):
            def idx_grp(j, carry):
                sl = pl.ds(j0 + j * NLANE, NLANE)
                idx_v[pl.ds(j * NLANE, NLANE)] = (
                    xd_st[0, sl] * NEMB + yd_st[0, sl])
                return carry

            lax.fori_loop(0, SLAB // NLANE, idx_grp, 0)

            def group(g, carry):
                iv = idx_v[pl.ds(g * NLANE, NLANE)]
                for i in range(NLANE):
                    pltpu.async_copy(comb_sp.at[iv[i]],
                                     rows.at[buf, g * NLANE + i], sem_g)
                return carry

            lax.fori_loop(0, SLAB // NLANE, group, 0)
            pltpu.make_async_copy(comb_sp.at[pl.ds(0, SLAB)], rows.at[buf],
                                  sem_g).wait()

        def write_wait():
            pltpu.make_async_copy(rows.at[0],
                                  out_hbm.at[0, pl.ds(0, SLAB)],
                                  sem_w).wait()

        def body(k, carry):
            u = wid + k * NW

            @pl.when(u < n_units)
            def _():
                plane = u // 2
                j0 = (u % 2) * SLAB
                buf = k % 2
                pltpu.sync_copy(xd_hbm.at[plane], xd_st)
                pltpu.sync_copy(yd_hbm.at[plane], yd_st)

                @pl.when(k >= 2)
                def _():
                    write_wait()

                fill(j0, buf)
                pltpu.make_async_copy(
                    rows.at[buf], out_hbm.at[plane, pl.ds(j0, SLAB)],
                    sem_w).start()

            return carry

        lax.fori_loop(0, kmax, body, 0)
        write_wait()
        write_wait()

    return sc_fn


def _tail_tc(sc_out, x_table, y_table, xcol, ycol):
    s = sc_out.shape[0]

    def body(prev_ref, xt_ref, yt_ref, xc_ref, yc_ref, out_ref):
        del prev_ref
        emb = jax.lax.iota(jnp.int32, NEMB)[None, :]
        ohx = (xc_ref[...] == emb).astype(jnp.float32)
        ohy = (yc_ref[...] == emb).astype(jnp.float32)
        xe = jnp.dot(ohx, xt_ref[...], preferred_element_type=jnp.float32,
                     precision=jax.lax.Precision.HIGHEST)
        ye = jnp.dot(ohy, yt_ref[...], preferred_element_type=jnp.float32,
                     precision=jax.lax.Precision.HIGHEST)
        out_ref[:, 0, :] = jnp.concatenate([xe, ye], axis=-1)

    return pl.pallas_call(
        body,
        grid=(1,),
        in_specs=[
            pl.BlockSpec(memory_space=pltpu.MemorySpace.HBM),
            pl.BlockSpec((NEMB, HALF), lambda i: (0, 0)),
            pl.BlockSpec((NEMB, HALF), lambda i: (0, 0)),
            pl.BlockSpec((s, 1), lambda i: (0, 0)),
            pl.BlockSpec((s, 1), lambda i: (0, 0)),
        ],
        out_specs=pl.BlockSpec((s, 8, 2 * HALF),
                               lambda i: (0, (s - 1) // 8, 0)),
        out_shape=jax.ShapeDtypeStruct((s, s, 2 * HALF), jnp.float32),
        input_output_aliases={0: 0},
    )(sc_out, x_table, y_table, xcol, ycol)


def kernel(x_table, y_table, x_distances, y_distances):
    s = x_distances.shape[0]
    s_pad = -(-s // 8) * 8

    xdp = jnp.pad(x_distances, ((0, 0), (0, s_pad - s)))[:, None, :]
    ydp = jnp.pad(y_distances, ((0, 0), (0, s_pad - s)))[:, None, :]

    sc_out = _build_sc_call(s, s_pad)(
        x_table.reshape(-1), y_table.reshape(-1), xdp, ydp)
    return _tail_tc(sc_out, x_table, y_table,
                    x_distances[:, s - 1][:, None],
                    y_distances[:, s - 1][:, None])


# speedup vs baseline: 1.0197x; 1.0197x over previous
import functools

import jax
import jax.numpy as jnp
from jax import lax
from jax.experimental import pallas as pl
from jax.experimental.pallas import tpu as pltpu
from jax.experimental.pallas import tpu_sc as plsc

HALF = 192
NEMB = 32
NW = 32
SLAB = 128
NLANE = 16
RPT = (NEMB * NEMB) // 16


def _build_sc_call(s, s_pad):
    n_units = s * 2
    kmax = -(-n_units // NW)
    mesh = plsc.VectorSubcoreMesh(core_axis_name="c", subcore_axis_name="s")

    @functools.partial(
        pl.kernel,
        mesh=mesh,
        out_type=jax.ShapeDtypeStruct((s, s, 2 * HALF), jnp.float32),
        scratch_types=[
            pltpu.VMEM((2 * HALF,), jnp.float32),
            pltpu.VMEM((NEMB * HALF,), jnp.float32),
            pltpu.VMEM((2, 2, s_pad), jnp.int32),
            pltpu.VMEM((SLAB,), jnp.int32),
            pltpu.VMEM((2, SLAB, 2 * HALF), jnp.float32),
            pltpu.VMEM_SHARED((NEMB * NEMB, 2 * HALF), jnp.float32),
            pltpu.SemaphoreType.DMA,
            pltpu.SemaphoreType.DMA,
            pltpu.SemaphoreType.DMA,
        ],
    )
    def sc_fn(xt_hbm, yt_hbm, idx_hbm, out_hbm, xt_v, yt_v,
              st, idx_v, rows, comb_sp, sem_g, sem_w, sem_s):
        sid = lax.axis_index("s")
        wid = sid * 2 + lax.axis_index("c")
        pltpu.sync_copy(xt_hbm.at[pl.ds(sid * 2 * HALF, 2 * HALF)], xt_v)
        pltpu.sync_copy(yt_hbm, yt_v)

        def brow(i, carry):
            a = i // NEMB
            b = i % NEMB
            for k in range(HALF // NLANE):
                rows[0, i, pl.ds(k * NLANE, NLANE)] = (
                    xt_v[pl.ds(a * HALF + k * NLANE, NLANE)])
                rows[0, i, pl.ds(HALF + k * NLANE, NLANE)] = (
                    yt_v[pl.ds(b * HALF + k * NLANE, NLANE)])
            return carry

        lax.fori_loop(0, RPT, brow, 0)
        pltpu.sync_copy(rows.at[0, pl.ds(0, RPT)],
                        comb_sp.at[pl.ds(sid * RPT, RPT)])

        def stage(k):
            plane = (wid + k * NW) // 2
            return pltpu.make_async_copy(idx_hbm.at[plane], st.at[k % 2],
                                         sem_s)

        def stage_wait():
            pltpu.make_async_copy(idx_hbm.at[0], st.at[0], sem_s).wait()

        plsc.subcore_barrier()
        stage(0).start()

        def fill(j0, buf, buf2):
            def idx_grp(j, carry):
                sl = pl.ds(j0 + j * NLANE, NLANE)
                idx_v[pl.ds(j * NLANE, NLANE)] = (
                    st[buf2, 0, sl] * NEMB + st[buf2, 1, sl])
                return carry

            lax.fori_loop(0, SLAB // NLANE, idx_grp, 0)

            def group(g, carry):
                iv = idx_v[pl.ds(g * NLANE, NLANE)]
                for i in range(NLANE):
                    pltpu.async_copy(comb_sp.at[iv[i]],
                                     rows.at[buf, g * NLANE + i], sem_g)
                return carry

            lax.fori_loop(0, SLAB // NLANE, group, 0)
            pltpu.make_async_copy(comb_sp.at[pl.ds(0, SLAB)], rows.at[buf],
                                  sem_g).wait()

        def write_wait():
            pltpu.make_async_copy(rows.at[0],
                                  out_hbm.at[0, pl.ds(0, SLAB)],
                                  sem_w).wait()

        def body(k, carry):
            u = wid + k * NW

            @pl.when(u < n_units)
            def _():
                plane = u // 2
                j0 = (u % 2) * SLAB
                buf = k % 2
                stage_wait()

                @pl.when(u + NW < n_units)
                def _():
                    stage(k + 1).start()

                @pl.when(k >= 2)
                def _():
                    write_wait()

                fill(j0, buf, k % 2)
                pltpu.make_async_copy(
                    rows.at[buf], out_hbm.at[plane, pl.ds(j0, SLAB)],
                    sem_w).start()

            return carry

        lax.fori_loop(0, kmax, body, 0)
        write_wait()
        write_wait()

    return sc_fn


def _tail_tc(sc_out, x_table, y_table, xcol, ycol):
    s = sc_out.shape[0]

    def body(prev_ref, xt_ref, yt_ref, xc_ref, yc_ref, out_ref):
        del prev_ref
        emb = jax.lax.iota(jnp.int32, NEMB)[None, :]
        ohx = (xc_ref[...] == emb).astype(jnp.float32)
        ohy = (yc_ref[...] == emb).astype(jnp.float32)
        xe = jnp.dot(ohx, xt_ref[...], preferred_element_type=jnp.float32,
                     precision=jax.lax.Precision.HIGHEST)
        ye = jnp.dot(ohy, yt_ref[...], preferred_element_type=jnp.float32,
                     precision=jax.lax.Precision.HIGHEST)
        out_ref[:, 0, :] = jnp.concatenate([xe, ye], axis=-1)

    return pl.pallas_call(
        body,
        grid=(1,),
        in_specs=[
            pl.BlockSpec(memory_space=pltpu.MemorySpace.HBM),
            pl.BlockSpec((NEMB, HALF), lambda i: (0, 0)),
            pl.BlockSpec((NEMB, HALF), lambda i: (0, 0)),
            pl.BlockSpec((s, 1), lambda i: (0, 0)),
            pl.BlockSpec((s, 1), lambda i: (0, 0)),
        ],
        out_specs=pl.BlockSpec((s, 8, 2 * HALF),
                               lambda i: (0, (s - 1) // 8, 0)),
        out_shape=jax.ShapeDtypeStruct((s, s, 2 * HALF), jnp.float32),
        input_output_aliases={0: 0},
    )(sc_out, x_table, y_table, xcol, ycol)


def kernel(x_table, y_table, x_distances, y_distances):
    s = x_distances.shape[0]
    s_pad = -(-s // 8) * 8

    idxsrc = jnp.stack([
        jnp.pad(x_distances, ((0, 0), (0, s_pad - s))),
        jnp.pad(y_distances, ((0, 0), (0, s_pad - s)))], axis=1)

    sc_out = _build_sc_call(s, s_pad)(
        x_table.reshape(-1), y_table.reshape(-1), idxsrc)
    return _tail_tc(sc_out, x_table, y_table,
                    x_distances[:, s - 1][:, None],
                    y_distances[:, s - 1][:, None])


# speedup vs baseline: 1.1358x; 1.1139x over previous
import functools

import jax
import jax.numpy as jnp
from jax import lax
from jax.experimental import pallas as pl
from jax.experimental.pallas import tpu as pltpu
from jax.experimental.pallas import tpu_sc as plsc

HALF = 192
NEMB = 32
NW = 32
SLAB = 64
NBUF = 3
NLANE = 16
RPT = (NEMB * NEMB) // 16


def _build_sc_call(s, s_pad):
    n_units = s * 4
    kmax = -(-n_units // NW)
    mesh = plsc.VectorSubcoreMesh(core_axis_name="c", subcore_axis_name="s")

    @functools.partial(
        pl.kernel,
        mesh=mesh,
        out_type=jax.ShapeDtypeStruct((s, s, 2 * HALF), jnp.float32),
        scratch_types=[
            pltpu.VMEM((2 * HALF,), jnp.float32),
            pltpu.VMEM((NEMB * HALF,), jnp.float32),
            pltpu.VMEM((2, 2, s_pad), jnp.int32),
            pltpu.VMEM((SLAB,), jnp.int32),
            pltpu.VMEM((NBUF, SLAB, 2 * HALF), jnp.float32),
            pltpu.VMEM_SHARED((NEMB * NEMB, 2 * HALF), jnp.float32),
            pltpu.SemaphoreType.DMA,
            pltpu.SemaphoreType.DMA,
            pltpu.SemaphoreType.DMA,
        ],
    )
    def sc_fn(xt_hbm, yt_hbm, idx_hbm, out_hbm, xt_v, yt_v,
              st, idx_v, rows, comb_sp, sem_g, sem_w, sem_s):
        sid = lax.axis_index("s")
        wid = sid * 2 + lax.axis_index("c")
        pltpu.sync_copy(xt_hbm.at[pl.ds(sid * 2 * HALF, 2 * HALF)], xt_v)
        pltpu.sync_copy(yt_hbm, yt_v)

        def brow(i, carry):
            a = i // NEMB
            b = i % NEMB
            for k in range(HALF // NLANE):
                rows[0, i, pl.ds(k * NLANE, NLANE)] = (
                    xt_v[pl.ds(a * HALF + k * NLANE, NLANE)])
                rows[0, i, pl.ds(HALF + k * NLANE, NLANE)] = (
                    yt_v[pl.ds(b * HALF + k * NLANE, NLANE)])
            return carry

        lax.fori_loop(0, RPT, brow, 0)
        pltpu.sync_copy(rows.at[0, pl.ds(0, RPT)],
                        comb_sp.at[pl.ds(sid * RPT, RPT)])

        def stage(k):
            plane = (wid + k * NW) // 4
            return pltpu.make_async_copy(idx_hbm.at[plane], st.at[k % 2],
                                         sem_s)

        def stage_wait():
            pltpu.make_async_copy(idx_hbm.at[0], st.at[0], sem_s).wait()

        plsc.subcore_barrier()
        stage(0).start()

        def fill(j0, buf, buf2):
            def idx_grp(j, carry):
                sl = pl.ds(j0 + j * NLANE, NLANE)
                idx_v[pl.ds(j * NLANE, NLANE)] = (
                    st[buf2, 0, sl] * NEMB + st[buf2, 1, sl])
                return carry

            lax.fori_loop(0, SLAB // NLANE, idx_grp, 0)

            def group(g, carry):
                iv = idx_v[pl.ds(g * NLANE, NLANE)]
                for i in range(NLANE):
                    pltpu.async_copy(comb_sp.at[iv[i]],
                                     rows.at[buf, g * NLANE + i], sem_g)
                return carry

            lax.fori_loop(0, SLAB // NLANE, group, 0)
            pltpu.make_async_copy(comb_sp.at[pl.ds(0, SLAB)], rows.at[buf],
                                  sem_g).wait()

        def write_wait():
            pltpu.make_async_copy(rows.at[0],
                                  out_hbm.at[0, pl.ds(0, SLAB)],
                                  sem_w).wait()

        def body(k, carry):
            u = wid + k * NW

            @pl.when(u < n_units)
            def _():
                plane = u // 4
                j0 = (u % 4) * SLAB
                buf = k % NBUF
                stage_wait()

                @pl.when(u + NW < n_units)
                def _():
                    stage(k + 1).start()

                @pl.when(k >= NBUF)
                def _():
                    write_wait()

                fill(j0, buf, k % 2)
                pltpu.make_async_copy(
                    rows.at[buf], out_hbm.at[plane, pl.ds(j0, SLAB)],
                    sem_w).start()

            return carry

        lax.fori_loop(0, kmax, body, 0)
        for _ in range(NBUF):
            write_wait()

    return sc_fn


def _tail_tc(sc_out, x_table, y_table, xcol, ycol):
    s = sc_out.shape[0]

    def body(prev_ref, xt_ref, yt_ref, xc_ref, yc_ref, out_ref):
        del prev_ref
        emb = jax.lax.iota(jnp.int32, NEMB)[None, :]
        ohx = (xc_ref[...] == emb).astype(jnp.float32)
        ohy = (yc_ref[...] == emb).astype(jnp.float32)
        xe = jnp.dot(ohx, xt_ref[...], preferred_element_type=jnp.float32,
                     precision=jax.lax.Precision.HIGHEST)
        ye = jnp.dot(ohy, yt_ref[...], preferred_element_type=jnp.float32,
                     precision=jax.lax.Precision.HIGHEST)
        out_ref[:, 0, :] = jnp.concatenate([xe, ye], axis=-1)

    return pl.pallas_call(
        body,
        grid=(1,),
        in_specs=[
            pl.BlockSpec(memory_space=pltpu.MemorySpace.HBM),
            pl.BlockSpec((NEMB, HALF), lambda i: (0, 0)),
            pl.BlockSpec((NEMB, HALF), lambda i: (0, 0)),
            pl.BlockSpec((s, 1), lambda i: (0, 0)),
            pl.BlockSpec((s, 1), lambda i: (0, 0)),
        ],
        out_specs=pl.BlockSpec((s, 8, 2 * HALF),
                               lambda i: (0, (s - 1) // 8, 0)),
        out_shape=jax.ShapeDtypeStruct((s, s, 2 * HALF), jnp.float32),
        input_output_aliases={0: 0},
    )(sc_out, x_table, y_table, xcol, ycol)


def kernel(x_table, y_table, x_distances, y_distances):
    s = x_distances.shape[0]
    s_pad = -(-s // 8) * 8

    idxsrc = jnp.stack([
        jnp.pad(x_distances, ((0, 0), (0, s_pad - s))),
        jnp.pad(y_distances, ((0, 0), (0, s_pad - s)))], axis=1)

    sc_out = _build_sc_call(s, s_pad)(
        x_table.reshape(-1), y_table.reshape(-1), idxsrc)
    return _tail_tc(sc_out, x_table, y_table,
                    x_distances[:, s - 1][:, None],
                    y_distances[:, s - 1][:, None])
